# baseline (device time: 163507 ns/iter reference)
import jax
import jax.numpy as jnp
from jax import lax
from jax.experimental import pallas as pl
from jax.experimental.pallas import tpu as pltpu

N_DEV = 4
B, SQ, SKV_G, HQ_G, DH = 2, 512, 2048, 32, 64
H_LOC = HQ_G // N_DEV
SKV_LOC = SKV_G // N_DEV
DM = 768
DQ_LOC = H_LOC * DH
NA = SKV_LOC + 128
GR = 32
MESH = pl.DeviceIdType.MESH

R_K0, R_V0, R_K1, R_V1 = 0, 1, 2, 3
R_Q32 = 4
R_NUM = 7
R_DM = 10
R_AR = 13
S_KV = 0
S_Q32 = 6
S_NUM = 9
S_DM = 12
S_AR = 15


def _body(x_ref, wq_ref, k_ref, v_ref, wo_ref, out_ref,
          ka_ref, va_ref, q_ref, q32s_ref, q32g_ref,
          numb_ref, dm_ref, nstat_ref, dmstat_ref,
          ctx_ref, ar_ref, send_sems, recv_sems):
    me = lax.axis_index("i")

    barrier = pltpu.get_barrier_semaphore()
    for s in range(1, N_DEV):
        pl.semaphore_signal(barrier, inc=1, device_id=((me + s) % N_DEV,),
                            device_id_type=MESH)
    pl.semaphore_wait(barrier, N_DEV - 1)

    k0_cp, k1_cp = [], []
    for s in range(1, N_DEV):
        peer = (me + s) % N_DEV
        hsl = pl.ds(peer * DQ_LOC, DQ_LOC)
        k0_cp.append(pltpu.make_async_remote_copy(
            src_ref=k_ref.at[:, :, hsl], dst_ref=ka_ref.at[:, 0:SKV_LOC, :],
            send_sem=send_sems.at[S_KV + s - 1], recv_sem=recv_sems.at[R_K0],
            device_id=(peer,), device_id_type=MESH))
        k0_cp.append(pltpu.make_async_remote_copy(
            src_ref=v_ref.at[:, :, hsl], dst_ref=va_ref.at[:, 0:SKV_LOC, :],
            send_sem=send_sems.at[S_KV + 3 + s - 1], recv_sem=recv_sems.at[R_V0],
            device_id=(peer,), device_id_type=MESH))
        k1_cp.append(pltpu.make_async_remote_copy(
            src_ref=k_ref.at[:, 0:128, hsl], dst_ref=ka_ref.at[:, SKV_LOC:NA, :],
            send_sem=send_sems.at[S_KV + s - 1], recv_sem=recv_sems.at[R_K1],
            device_id=(peer,), device_id_type=MESH))
        k1_cp.append(pltpu.make_async_remote_copy(
            src_ref=v_ref.at[:, 0:128, hsl], dst_ref=va_ref.at[:, SKV_LOC:NA, :],
            send_sem=send_sems.at[S_KV + 3 + s - 1], recv_sem=recv_sems.at[R_V1],
            device_id=(peer,), device_id_type=MESH))

    @pl.when(me == 0)
    def _():
        for cp in k0_cp:
            cp.start()

    @pl.when(me == 1)
    def _():
        for cp in k1_cp:
            cp.start()

    my_hsl = pl.ds(me * DQ_LOC, DQ_LOC)

    @pl.when(me == 0)
    def _():
        ka_ref[:, 0:SKV_LOC, :] = k_ref[:, :, my_hsl]
        va_ref[:, 0:SKV_LOC, :] = v_ref[:, :, my_hsl]

    @pl.when(me == 1)
    def _():
        ka_ref[:, SKV_LOC:NA, :] = k_ref[:, 0:128, my_hsl]
        va_ref[:, SKV_LOC:NA, :] = v_ref[:, 0:128, my_hsl]

    x2d = x_ref[...].reshape(B * SQ, DM)
    qf = lax.dot_general(x2d, wq_ref[...], (((1,), (0,)), ((), ())),
                         preferred_element_type=jnp.float32)
    q_ref[...] = (qf * 0.125).astype(jnp.bfloat16)

    q32s_ref[0:GR] = q_ref[0:GR]
    q32s_ref[GR:2 * GR] = q_ref[SQ:SQ + GR]
    q32g_ref[0] = q32s_ref[...]
    q32_cp = []
    for s in range(1, N_DEV):
        peer = (me + s) % N_DEV
        cp = pltpu.make_async_remote_copy(
            src_ref=q32s_ref, dst_ref=q32g_ref.at[N_DEV - s],
            send_sem=send_sems.at[S_Q32 + s - 1],
            recv_sem=recv_sems.at[R_Q32 + s - 1],
            device_id=(peer,), device_id_type=MESH)
        cp.start()
        q32_cp.append(cp)
    for cp in q32_cp:
        cp.wait_recv()

    kcol = lax.broadcasted_iota(jnp.int32, (1, SKV_LOC), 1) + me * SKV_LOC
    vmask = kcol >= NA
    neg = jnp.float32(-1e9)
    slot = [(jnp.int32(c) - me) % N_DEV for c in range(N_DEV)]

    def dot_t(a, bm):
        return lax.dot_general(a, bm, (((1,), (1,)), ((), ())),
                               preferred_element_type=jnp.float32)

    def dot_n(a, bm):
        return lax.dot_general(a, bm, (((1,), (0,)), ((), ())),
                               preferred_element_type=jnp.float32)

    vmask3 = vmask[None]
    for g in range(N_DEV):
        qg = q32g_ref[pl.ds(slot[g], 1)][0]
        for b in range(B):
            q3 = qg[b * GR:(b + 1) * GR, :].reshape(GR, H_LOC, DH)
            k3 = k_ref[b, :, g * DQ_LOC:(g + 1) * DQ_LOC].reshape(
                SKV_LOC, H_LOC, DH)
            v3 = v_ref[b, :, g * DQ_LOC:(g + 1) * DQ_LOC].reshape(
                SKV_LOC, H_LOC, DH)
            sb3 = lax.dot_general(q3, k3, (((2,), (2,)), ((1,), (1,))),
                                  preferred_element_type=jnp.float32)
            sb3 = jnp.where(vmask3, sb3, neg)
            mb3 = jnp.max(sb3, axis=2, keepdims=True)
            eb3 = jnp.exp(sb3 - mb3).astype(jnp.bfloat16)
            db3 = jnp.sum(eb3.astype(jnp.float32), axis=2, keepdims=True)
            nb3 = lax.dot_general(
                eb3, v3, (((2,), (0,)), ((0,), (1,))),
                preferred_element_type=jnp.float32)
            numb_ref[g, b * GR:(b + 1) * GR, :] = (
                nb3.transpose(1, 0, 2).reshape(GR, DQ_LOC).astype(jnp.bfloat16))
            dm_ref[g, 0, b * GR:(b + 1) * GR, :] = db3[:, :, 0].transpose(1, 0)
            dm_ref[g, 1, b * GR:(b + 1) * GR, :] = mb3[:, :, 0].transpose(1, 0)

    nstat_ref[pl.ds(0, 1)] = numb_ref[pl.ds(me, 1)]
    dmstat_ref[pl.ds(0, 1)] = dm_ref[pl.ds(me, 1)]
    st_cp = []
    for s in range(1, N_DEV):
        peer = (me + s) % N_DEV
        cp = pltpu.make_async_remote_copy(
            src_ref=numb_ref.at[pl.ds(peer, 1)],
            dst_ref=nstat_ref.at[pl.ds(N_DEV - s, 1)],
            send_sem=send_sems.at[S_NUM + s - 1],
            recv_sem=recv_sems.at[R_NUM + s - 1],
            device_id=(peer,), device_id_type=MESH)
        cp.start()
        st_cp.append(cp)
        cp = pltpu.make_async_remote_copy(
            src_ref=dm_ref.at[pl.ds(peer, 1)],
            dst_ref=dmstat_ref.at[pl.ds(N_DEV - s, 1)],
            send_sem=send_sems.at[S_DM + s - 1],
            recv_sem=recv_sems.at[R_DM + s - 1],
            device_id=(peer,), device_id_type=MESH)
        cp.start()
        st_cp.append(cp)

    @pl.when(me != 0)
    def _():
        k0_cp[0].wait_recv()
        k0_cp[1].wait_recv()

    @pl.when(me != 1)
    def _():
        k1_cp[0].wait_recv()
        k1_cp[1].wait_recv()

    for cp in st_cp:
        cp.wait_recv()

    qi = lax.broadcasted_iota(jnp.int32, (SQ, NA), 0)
    ki = lax.broadcasted_iota(jnp.int32, (SQ, NA), 1)
    mask_a = (jnp.abs(qi - ki) <= 128) | (ki < 32) | (qi < 32)
    zctx = jnp.zeros((SQ - GR, DH), jnp.float32)

    ar_cp = []
    for b in range(B):
        for h in range(H_LOC):
            hs = slice(h * DH, (h + 1) * DH)
            qh = q_ref[pl.ds(b * SQ, SQ), hs]
            sa = dot_t(qh, ka_ref[b, :, hs])
            sa = jnp.where(mask_a, sa, neg)
            ma = jnp.max(sa, axis=1, keepdims=True)
            m32 = ma[:GR]
            mb_t = [dmstat_ref[t, 1, pl.ds(b * GR, GR), h:h + 1]
                    for t in range(N_DEV)]
            for mb in mb_t:
                m32 = jnp.maximum(m32, mb)
            m = jnp.concatenate([m32, ma[GR:]], axis=0)
            ea = jnp.exp(sa - m).astype(jnp.bfloat16)
            den = jnp.sum(ea.astype(jnp.float32), axis=1, keepdims=True)
            na = dot_n(ea, va_ref[b, :, hs])
            num32 = jnp.zeros((GR, DH), jnp.float32)
            den32 = jnp.zeros((GR, 1), jnp.float32)
            for t in range(N_DEV):
                scale = jnp.exp(mb_t[t] - m32)
                db_t = dmstat_ref[t, 0, pl.ds(b * GR, GR), h:h + 1]
                nb_t = nstat_ref[t, pl.ds(b * GR, GR), hs].astype(jnp.float32)
                den32 = den32 + db_t * scale
                num32 = num32 + nb_t * scale
            num = na + jnp.concatenate([num32, zctx], axis=0)
            den = den + jnp.concatenate(
                [den32, jnp.zeros((SQ - GR, 1), jnp.float32)], axis=0)
            ctx_ref[pl.ds(b * SQ, SQ), hs] = (num / den).astype(jnp.bfloat16)

        bsl = pl.ds(b * SQ, SQ)
        partial_b = lax.dot_general(ctx_ref[bsl], wo_ref[...],
                                    (((1,), (0,)), ((), ())),
                                    preferred_element_type=jnp.float32)
        ar_ref[0, bsl] = partial_b.astype(jnp.bfloat16)
        for s in range(1, N_DEV):
            peer = (me + s) % N_DEV
            cp = pltpu.make_async_remote_copy(
                src_ref=ar_ref.at[0, bsl], dst_ref=ar_ref.at[N_DEV - s, bsl],
                send_sem=send_sems.at[S_AR + 3 * b + s - 1],
                recv_sem=recv_sems.at[R_AR + 3 * b + s - 1],
                device_id=(peer,), device_id_type=MESH)
            cp.start()
            ar_cp.append(cp)

    for cp in ar_cp:
        cp.wait_recv()

    out_ref[...] = (ar_ref[0].astype(jnp.float32)
                    + ar_ref[1].astype(jnp.float32)).reshape(B, SQ, DM)
    out_ref[...] = (out_ref[...].reshape(B * SQ, DM)
                    + ar_ref[2].astype(jnp.float32)
                    + ar_ref[3].astype(jnp.float32)).reshape(B, SQ, DM)

    @pl.when(me == 0)
    def _():
        for cp in k0_cp:
            cp.wait_send()

    @pl.when(me == 1)
    def _():
        for cp in k1_cp:
            cp.wait_send()

    for cp in q32_cp + st_cp + ar_cp:
        cp.wait_send()


def kernel(x, Wq, K_ext, V_ext, Wo):
    xb = x.astype(jnp.bfloat16)
    wqb = Wq.astype(jnp.bfloat16)
    wob = Wo.astype(jnp.bfloat16)
    kb = K_ext.reshape(B, SKV_LOC, HQ_G * DH).astype(jnp.bfloat16)
    vb = V_ext.reshape(B, SKV_LOC, HQ_G * DH).astype(jnp.bfloat16)

    out_shape = jax.ShapeDtypeStruct((B, SQ, DM), jnp.float32)
    return pl.pallas_call(
        _body,
        out_shape=out_shape,
        in_specs=[pl.BlockSpec(memory_space=pltpu.VMEM)] * 5,
        out_specs=pl.BlockSpec(memory_space=pltpu.VMEM),
        scratch_shapes=[
            pltpu.VMEM((B, NA, DQ_LOC), jnp.bfloat16),
            pltpu.VMEM((B, NA, DQ_LOC), jnp.bfloat16),
            pltpu.VMEM((B * SQ, DQ_LOC), jnp.bfloat16),
            pltpu.VMEM((B * GR, DQ_LOC), jnp.bfloat16),
            pltpu.VMEM((N_DEV, B * GR, DQ_LOC), jnp.bfloat16),
            pltpu.VMEM((N_DEV, B * GR, DQ_LOC), jnp.bfloat16),
            pltpu.VMEM((N_DEV, 2, B * GR, H_LOC), jnp.float32),
            pltpu.VMEM((N_DEV, B * GR, DQ_LOC), jnp.bfloat16),
            pltpu.VMEM((N_DEV, 2, B * GR, H_LOC), jnp.float32),
            pltpu.VMEM((B * SQ, DQ_LOC), jnp.bfloat16),
            pltpu.VMEM((N_DEV, B * SQ, DM), jnp.bfloat16),
            pltpu.SemaphoreType.DMA((21,)),
            pltpu.SemaphoreType.DMA((19,)),
        ],
        compiler_params=pltpu.CompilerParams(collective_id=0),
    )(xb, wqb, kb, vb, wob)


# device time: 122603 ns/iter; 1.3336x vs baseline; 1.3336x over previous
import os

import jax
import jax.numpy as jnp
from jax import lax
from jax.experimental import pallas as pl
from jax.experimental.pallas import tpu as pltpu

ABLATE_AR = os.environ.get("ABLATE_AR") == "1"
ABLATE_ATTN = os.environ.get("ABLATE_ATTN") == "1"

N_DEV = 4
B, SQ, SKV_G, HQ_G, DH = 2, 512, 2048, 32, 64
H_LOC = HQ_G // N_DEV
SKV_LOC = SKV_G // N_DEV
DM = 768
DQ_LOC = H_LOC * DH


def _body(x_ref, wq_ref, k_ref, v_ref, wo_ref, out_ref,
          kg_ref, vg_ref, ar_ref, q_ref, ctx_ref, send_sems, recv_sems):
    me = lax.axis_index("i")

    barrier = pltpu.get_barrier_semaphore()
    for s in range(1, N_DEV):
        pl.semaphore_signal(
            barrier, inc=1,
            device_id=((me + s) % N_DEV,),
            device_id_type=pl.DeviceIdType.MESH,
        )
    pl.semaphore_wait(barrier, N_DEV - 1)

    kg_ref[0] = k_ref[:, :, pl.ds(me * DQ_LOC, DQ_LOC)]
    vg_ref[0] = v_ref[:, :, pl.ds(me * DQ_LOC, DQ_LOC)]

    copies = []
    for s in range(1, N_DEV):
        peer = (me + s) % N_DEV
        kcp = pltpu.make_async_remote_copy(
            src_ref=k_ref.at[:, :, pl.ds(peer * DQ_LOC, DQ_LOC)],
            dst_ref=kg_ref.at[N_DEV - s],
            send_sem=send_sems.at[s - 1],
            recv_sem=recv_sems.at[s - 1],
            device_id=(peer,),
            device_id_type=pl.DeviceIdType.MESH,
        )
        vcp = pltpu.make_async_remote_copy(
            src_ref=v_ref.at[:, :, pl.ds(peer * DQ_LOC, DQ_LOC)],
            dst_ref=vg_ref.at[N_DEV - s],
            send_sem=send_sems.at[3 + s - 1],
            recv_sem=recv_sems.at[3 + s - 1],
            device_id=(peer,),
            device_id_type=pl.DeviceIdType.MESH,
        )
        kcp.start()
        vcp.start()
        copies.append(kcp)
        copies.append(vcp)

    x2d = x_ref[...].reshape(B * SQ, DM)
    qf = lax.dot_general(
        x2d, wq_ref[...], (((1,), (0,)), ((), ())),
        preferred_element_type=jnp.float32,
    )
    q_ref[...] = (qf * 0.125).astype(jnp.bfloat16)

    NA = SKV_LOC + 128
    NB = SKV_G - NA
    GR = 32
    qi = lax.broadcasted_iota(jnp.int32, (SQ, NA), 0)
    ki = lax.broadcasted_iota(jnp.int32, (SQ, NA), 1)
    mask_a = (jnp.abs(qi - ki) <= 128) | (ki < 32) | (qi < 32)
    neg = jnp.float32(-1e9)
    minf = jnp.full((SQ - GR, 1), -1e30, jnp.float32)
    zcol = jnp.zeros((SQ - GR, 1), jnp.float32)
    zctx = jnp.zeros((SQ - GR, DH), jnp.float32)

    for cp in copies:
        cp.wait_recv()

    slot = [(jnp.int32(c) - me) % N_DEV for c in range(N_DEV)]

    def dot_t(a, bm):
        return lax.dot_general(a, bm, (((1,), (1,)), ((), ())),
                               preferred_element_type=jnp.float32)

    def dot_n(a, bm):
        return lax.dot_general(a, bm, (((1,), (0,)), ((), ())),
                               preferred_element_type=jnp.float32)

    ar_copies = []
    for b in range(B):
        for h in range(H_LOC):
            if ABLATE_ATTN:
                break
            hs = slice(h * DH, (h + 1) * DH)
            qh = q_ref[pl.ds(b * SQ, SQ), hs]
            q32 = qh[:GR]
            kc = [kg_ref[pl.ds(slot[c], 1), b, :, hs][0] for c in range(N_DEV)]
            vc = [vg_ref[pl.ds(slot[c], 1), b, :, hs][0] for c in range(N_DEV)]
            sa = jnp.concatenate(
                [dot_t(qh, kc[0]), dot_t(qh, kc[1][:128])], axis=1)
            sa = jnp.where(mask_a, sa, neg)
            sb = jnp.concatenate(
                [dot_t(q32, kc[1][128:]), dot_t(q32, kc[2]),
                 dot_t(q32, kc[3])], axis=1)
            ma = jnp.max(sa, axis=1, keepdims=True)
            mb = jnp.max(sb, axis=1, keepdims=True)
            m = jnp.maximum(ma, jnp.concatenate([mb, minf], axis=0))
            ea = jnp.exp(sa - m).astype(jnp.bfloat16)
            eb = jnp.exp(sb - m[:GR]).astype(jnp.bfloat16)
            denom = (jnp.sum(ea.astype(jnp.float32), axis=1, keepdims=True)
                     + jnp.concatenate(
                         [jnp.sum(eb.astype(jnp.float32), axis=1,
                                  keepdims=True), zcol], axis=0))
            na = dot_n(ea[:, :SKV_LOC], vc[0]) + dot_n(ea[:, SKV_LOC:],
                                                       vc[1][:128])
            nb = (dot_n(eb[:, :SKV_LOC - 128], vc[1][128:])
                  + dot_n(eb[:, SKV_LOC - 128:2 * SKV_LOC - 128], vc[2])
                  + dot_n(eb[:, 2 * SKV_LOC - 128:], vc[3]))
            num = na + jnp.concatenate([nb, zctx], axis=0)
            ctx_ref[pl.ds(b * SQ, SQ), hs] = (num / denom).astype(jnp.bfloat16)

        bsl = pl.ds(b * SQ, SQ)
        partial_b = lax.dot_general(
            ctx_ref[bsl], wo_ref[...], (((1,), (0,)), ((), ())),
            preferred_element_type=jnp.float32,
        )
        ar_ref[0, bsl] = partial_b.astype(jnp.bfloat16)
        if not ABLATE_AR:
            for s in range(1, N_DEV):
                peer = (me + s) % N_DEV
                cp = pltpu.make_async_remote_copy(
                    src_ref=ar_ref.at[0, bsl],
                    dst_ref=ar_ref.at[N_DEV - s, bsl],
                    send_sem=send_sems.at[6 + 3 * b + s - 1],
                    recv_sem=recv_sems.at[6 + 3 * b + s - 1],
                    device_id=(peer,),
                    device_id_type=pl.DeviceIdType.MESH,
                )
                cp.start()
                ar_copies.append(cp)

    if ABLATE_AR:
        out_ref[...] = ar_ref[0].astype(jnp.float32).reshape(B, SQ, DM)
        for cp in copies:
            cp.wait_send()
        return

    for cp in ar_copies:
        cp.wait_recv()

    out_ref[...] = (ar_ref[0].astype(jnp.float32)
                    + ar_ref[1].astype(jnp.float32)).reshape(B, SQ, DM)
    out_ref[...] = (out_ref[...].reshape(B * SQ, DM)
                    + ar_ref[2].astype(jnp.float32)
                    + ar_ref[3].astype(jnp.float32)).reshape(B, SQ, DM)

    for cp in copies:
        cp.wait_send()
    for cp in ar_copies:
        cp.wait_send()


def kernel(x, Wq, K_ext, V_ext, Wo):
    xb = x.astype(jnp.bfloat16)
    wqb = Wq.astype(jnp.bfloat16)
    wob = Wo.astype(jnp.bfloat16)
    kb = K_ext.reshape(B, SKV_LOC, HQ_G * DH).astype(jnp.bfloat16)
    vb = V_ext.reshape(B, SKV_LOC, HQ_G * DH).astype(jnp.bfloat16)

    out_shape = jax.ShapeDtypeStruct((B, SQ, DM), jnp.float32)
    return pl.pallas_call(
        _body,
        out_shape=out_shape,
        in_specs=[pl.BlockSpec(memory_space=pltpu.VMEM)] * 5,
        out_specs=pl.BlockSpec(memory_space=pltpu.VMEM),
        scratch_shapes=[
            pltpu.VMEM((N_DEV, B, SKV_LOC, DQ_LOC), jnp.bfloat16),
            pltpu.VMEM((N_DEV, B, SKV_LOC, DQ_LOC), jnp.bfloat16),
            pltpu.VMEM((N_DEV, B * SQ, DM), jnp.bfloat16),
            pltpu.VMEM((B * SQ, DQ_LOC), jnp.bfloat16),
            pltpu.VMEM((B * SQ, DQ_LOC), jnp.bfloat16),
            pltpu.SemaphoreType.DMA((12,)),
            pltpu.SemaphoreType.DMA((12,)),
        ],
        compiler_params=pltpu.CompilerParams(collective_id=0),
    )(xb, wqb, kb, vb, wob)


# device time: 117761 ns/iter; 1.3885x vs baseline; 1.0411x over previous
import os

import jax
import jax.numpy as jnp
from jax import lax
from jax.experimental import pallas as pl
from jax.experimental.pallas import tpu as pltpu

ABLATE_AR = os.environ.get("ABLATE_AR") == "1"
ABLATE_ATTN = os.environ.get("ABLATE_ATTN") == "1"

N_DEV = 4
B, SQ, SKV_G, HQ_G, DH = 2, 512, 2048, 32, 64
H_LOC = HQ_G // N_DEV
SKV_LOC = SKV_G // N_DEV
DM = 768
DQ_LOC = H_LOC * DH
QTR = SQ // N_DEV


def _body(x_ref, wq_ref, k_ref, v_ref, wo_ref, out_ref,
          kg_ref, vg_ref, myp_ref, rs_ref, ag_ref, q_ref, ctx_ref,
          send_sems, recv_sems):
    me = lax.axis_index("i")

    barrier = pltpu.get_barrier_semaphore()
    for s in range(1, N_DEV):
        pl.semaphore_signal(
            barrier, inc=1,
            device_id=((me + s) % N_DEV,),
            device_id_type=pl.DeviceIdType.MESH,
        )
    pl.semaphore_wait(barrier, N_DEV - 1)

    kg_ref[0] = k_ref[:, :, pl.ds(me * DQ_LOC, DQ_LOC)]
    vg_ref[0] = v_ref[:, :, pl.ds(me * DQ_LOC, DQ_LOC)]

    copies = []
    for s in range(1, N_DEV):
        peer = (me + s) % N_DEV
        kcp = pltpu.make_async_remote_copy(
            src_ref=k_ref.at[:, :, pl.ds(peer * DQ_LOC, DQ_LOC)],
            dst_ref=kg_ref.at[N_DEV - s],
            send_sem=send_sems.at[s - 1],
            recv_sem=recv_sems.at[s - 1],
            device_id=(peer,),
            device_id_type=pl.DeviceIdType.MESH,
        )
        vcp = pltpu.make_async_remote_copy(
            src_ref=v_ref.at[:, :, pl.ds(peer * DQ_LOC, DQ_LOC)],
            dst_ref=vg_ref.at[N_DEV - s],
            send_sem=send_sems.at[3 + s - 1],
            recv_sem=recv_sems.at[3 + s - 1],
            device_id=(peer,),
            device_id_type=pl.DeviceIdType.MESH,
        )
        kcp.start()
        vcp.start()
        copies.append(kcp)
        copies.append(vcp)

    x2d = x_ref[...].reshape(B * SQ, DM)
    qf = lax.dot_general(
        x2d, wq_ref[...], (((1,), (0,)), ((), ())),
        preferred_element_type=jnp.float32,
    )
    q_ref[...] = (qf * 0.125).astype(jnp.bfloat16)

    NA = SKV_LOC + 128
    NB = SKV_G - NA
    GR = 32
    qi = lax.broadcasted_iota(jnp.int32, (SQ, NA), 0)
    ki = lax.broadcasted_iota(jnp.int32, (SQ, NA), 1)
    mask_a = (jnp.abs(qi - ki) <= 128) | (ki < 32) | (qi < 32)
    neg = jnp.float32(-1e9)
    minf = jnp.full((SQ - GR, 1), -1e30, jnp.float32)
    zcol = jnp.zeros((SQ - GR, 1), jnp.float32)
    zctx = jnp.zeros((SQ - GR, DH), jnp.float32)

    for cp in copies:
        cp.wait_recv()

    slot = [(jnp.int32(c) - me) % N_DEV for c in range(N_DEV)]

    def dot_t(a, bm):
        return lax.dot_general(a, bm, (((1,), (1,)), ((), ())),
                               preferred_element_type=jnp.float32)

    def dot_n(a, bm):
        return lax.dot_general(a, bm, (((1,), (0,)), ((), ())),
                               preferred_element_type=jnp.float32)

    rs_copies = []
    for b in range(B):
        for h in range(H_LOC):
            if ABLATE_ATTN:
                break
            hs = slice(h * DH, (h + 1) * DH)
            qh = q_ref[pl.ds(b * SQ, SQ), hs]
            q32 = qh[:GR]
            kc = [kg_ref[pl.ds(slot[c], 1), b, :, hs][0] for c in range(N_DEV)]
            vc = [vg_ref[pl.ds(slot[c], 1), b, :, hs][0] for c in range(N_DEV)]
            sa = jnp.concatenate(
                [dot_t(qh, kc[0]), dot_t(qh, kc[1][:128])], axis=1)
            sa = jnp.where(mask_a, sa, neg)
            sb = jnp.concatenate(
                [dot_t(q32, kc[1][128:]), dot_t(q32, kc[2]),
                 dot_t(q32, kc[3])], axis=1)
            ma = jnp.max(sa, axis=1, keepdims=True)
            mb = jnp.max(sb, axis=1, keepdims=True)
            m = jnp.maximum(ma, jnp.concatenate([mb, minf], axis=0))
            ea = jnp.exp(sa - m).astype(jnp.bfloat16)
            eb = jnp.exp(sb - m[:GR]).astype(jnp.bfloat16)
            denom = (jnp.sum(ea.astype(jnp.float32), axis=1, keepdims=True)
                     + jnp.concatenate(
                         [jnp.sum(eb.astype(jnp.float32), axis=1,
                                  keepdims=True), zcol], axis=0))
            na = dot_n(ea[:, :SKV_LOC], vc[0]) + dot_n(ea[:, SKV_LOC:],
                                                       vc[1][:128])
            nb = (dot_n(eb[:, :SKV_LOC - 128], vc[1][128:])
                  + dot_n(eb[:, SKV_LOC - 128:2 * SKV_LOC - 128], vc[2])
                  + dot_n(eb[:, 2 * SKV_LOC - 128:], vc[3]))
            num = na + jnp.concatenate([nb, zctx], axis=0)
            ctx_ref[pl.ds(b * SQ, SQ), hs] = (num / denom).astype(jnp.bfloat16)

        bsl = pl.ds(b * SQ, SQ)
        partial_b = lax.dot_general(
            ctx_ref[bsl], wo_ref[...], (((1,), (0,)), ((), ())),
            preferred_element_type=jnp.float32,
        )
        myp_ref[bsl] = partial_b.astype(jnp.bfloat16)
        if not ABLATE_AR:
            for s in range(1, N_DEV):
                peer = (me + s) % N_DEV
                cp = pltpu.make_async_remote_copy(
                    src_ref=myp_ref.at[pl.ds(b * SQ + peer * QTR, QTR)],
                    dst_ref=rs_ref.at[b, s - 1],
                    send_sem=send_sems.at[6 + 3 * b + s - 1],
                    recv_sem=recv_sems.at[6 + 3 * b + s - 1],
                    device_id=(peer,),
                    device_id_type=pl.DeviceIdType.MESH,
                )
                cp.start()
                rs_copies.append(cp)

    if ABLATE_AR:
        out_ref[...] = myp_ref[...].astype(jnp.float32).reshape(B, SQ, DM)
        for cp in copies:
            cp.wait_send()
        return

    ag_copies = []
    for b in range(B):
        for t in range(3):
            rs_copies[3 * b + t].wait_recv()
        myq = myp_ref[pl.ds(b * SQ + me * QTR, QTR)].astype(jnp.float32)
        for t in range(3):
            myq = myq + rs_ref[b, t].astype(jnp.float32)
        ag_ref[b, 3] = myq.astype(jnp.bfloat16)
        out_ref[b, pl.ds(me * QTR, QTR)] = myq
        for s in range(1, N_DEV):
            peer = (me + s) % N_DEV
            cp = pltpu.make_async_remote_copy(
                src_ref=ag_ref.at[b, 3],
                dst_ref=ag_ref.at[b, s - 1],
                send_sem=send_sems.at[12 + 3 * b + s - 1],
                recv_sem=recv_sems.at[12 + 3 * b + s - 1],
                device_id=(peer,),
                device_id_type=pl.DeviceIdType.MESH,
            )
            cp.start()
            ag_copies.append(cp)

    for b in range(B):
        for s in range(1, N_DEV):
            ag_copies[3 * b + s - 1].wait_recv()
            origin = (me - s) % N_DEV
            out_ref[b, pl.ds(origin * QTR, QTR)] = (
                ag_ref[b, s - 1].astype(jnp.float32))

    for cp in copies:
        cp.wait_send()
    for cp in rs_copies + ag_copies:
        cp.wait_send()


def kernel(x, Wq, K_ext, V_ext, Wo):
    xb = x.astype(jnp.bfloat16)
    wqb = Wq.astype(jnp.bfloat16)
    wob = Wo.astype(jnp.bfloat16)
    kb = K_ext.reshape(B, SKV_LOC, HQ_G * DH).astype(jnp.bfloat16)
    vb = V_ext.reshape(B, SKV_LOC, HQ_G * DH).astype(jnp.bfloat16)

    out_shape = jax.ShapeDtypeStruct((B, SQ, DM), jnp.float32)
    return pl.pallas_call(
        _body,
        out_shape=out_shape,
        in_specs=[pl.BlockSpec(memory_space=pltpu.VMEM)] * 5,
        out_specs=pl.BlockSpec(memory_space=pltpu.VMEM),
        scratch_shapes=[
            pltpu.VMEM((N_DEV, B, SKV_LOC, DQ_LOC), jnp.bfloat16),
            pltpu.VMEM((N_DEV, B, SKV_LOC, DQ_LOC), jnp.bfloat16),
            pltpu.VMEM((B * SQ, DM), jnp.bfloat16),
            pltpu.VMEM((B, 3, QTR, DM), jnp.bfloat16),
            pltpu.VMEM((B, 4, QTR, DM), jnp.bfloat16),
            pltpu.VMEM((B * SQ, DQ_LOC), jnp.bfloat16),
            pltpu.VMEM((B * SQ, DQ_LOC), jnp.bfloat16),
            pltpu.SemaphoreType.DMA((18,)),
            pltpu.SemaphoreType.DMA((18,)),
        ],
        compiler_params=pltpu.CompilerParams(collective_id=0),
    )(xb, wqb, kb, vb, wob)


# device time: 110820 ns/iter; 1.4754x vs baseline; 1.0626x over previous
import os

import jax
import jax.numpy as jnp
from jax import lax
from jax.experimental import pallas as pl
from jax.experimental.pallas import tpu as pltpu

ABLATE_AR = os.environ.get("ABLATE_AR") == "1"
ABLATE_ATTN = os.environ.get("ABLATE_ATTN") == "1"

N_DEV = 4
B, SQ, SKV_G, HQ_G, DH = 2, 512, 2048, 32, 64
H_LOC = HQ_G // N_DEV
SKV_LOC = SKV_G // N_DEV
DM = 768
DQ_LOC = H_LOC * DH
QTR = SQ // N_DEV


def _body(x_ref, wq_ref, k_ref, v_ref, wo_ref, out_ref,
          kg_ref, vg_ref, myp_ref, rs_ref, ag_ref, q_ref, ctx_ref,
          send_sems, recv_sems):
    me = lax.axis_index("i")

    barrier = pltpu.get_barrier_semaphore()
    for s in range(1, N_DEV):
        pl.semaphore_signal(
            barrier, inc=1,
            device_id=((me + s) % N_DEV,),
            device_id_type=pl.DeviceIdType.MESH,
        )
    pl.semaphore_wait(barrier, N_DEV - 1)

    kg_ref[0] = k_ref[:, :, pl.ds(me * DQ_LOC, DQ_LOC)]
    vg_ref[0] = v_ref[:, :, pl.ds(me * DQ_LOC, DQ_LOC)]

    copies = []
    for s in range(1, N_DEV):
        peer = (me + s) % N_DEV
        kcp = pltpu.make_async_remote_copy(
            src_ref=k_ref.at[:, :, pl.ds(peer * DQ_LOC, DQ_LOC)],
            dst_ref=kg_ref.at[N_DEV - s],
            send_sem=send_sems.at[s - 1],
            recv_sem=recv_sems.at[s - 1],
            device_id=(peer,),
            device_id_type=pl.DeviceIdType.MESH,
        )
        vcp = pltpu.make_async_remote_copy(
            src_ref=v_ref.at[:, :, pl.ds(peer * DQ_LOC, DQ_LOC)],
            dst_ref=vg_ref.at[N_DEV - s],
            send_sem=send_sems.at[3 + s - 1],
            recv_sem=recv_sems.at[3 + s - 1],
            device_id=(peer,),
            device_id_type=pl.DeviceIdType.MESH,
        )
        kcp.start()
        vcp.start()
        copies.append(kcp)
        copies.append(vcp)

    x2d = x_ref[...].reshape(B * SQ, DM)
    qf = lax.dot_general(
        x2d, wq_ref[...], (((1,), (0,)), ((), ())),
        preferred_element_type=jnp.float32,
    )
    q_ref[...] = (qf * 0.125).astype(jnp.bfloat16)

    NA = SKV_LOC + 128
    NB = SKV_G - NA
    GR = 32
    qi = lax.broadcasted_iota(jnp.int32, (SQ, NA), 0)
    ki = lax.broadcasted_iota(jnp.int32, (SQ, NA), 1)
    mask_a = (jnp.abs(qi - ki) <= 128) | (ki < 32) | (qi < 32)
    mask_a0 = mask_a[:, :SKV_LOC]
    mask_a1 = mask_a[:, SKV_LOC:]
    neg = jnp.float32(-1e9)
    minf = jnp.full((SQ - GR, 1), -1e30, jnp.float32)
    zcol = jnp.zeros((SQ - GR, 1), jnp.float32)
    zctx = jnp.zeros((SQ - GR, DH), jnp.float32)

    for cp in copies:
        cp.wait_recv()

    slot = [(jnp.int32(c) - me) % N_DEV for c in range(N_DEV)]

    def dot_t(a, bm):
        return lax.dot_general(a, bm, (((1,), (1,)), ((), ())),
                               preferred_element_type=jnp.float32)

    def dot_n(a, bm):
        return lax.dot_general(a, bm, (((1,), (0,)), ((), ())),
                               preferred_element_type=jnp.float32)

    rs_copies = []
    for b in range(B):
        for h in range(H_LOC):
            if ABLATE_ATTN:
                break
            hs = slice(h * DH, (h + 1) * DH)
            qh = q_ref[pl.ds(b * SQ, SQ), hs]
            q32 = qh[:GR]
            kc = [kg_ref[pl.ds(slot[c], 1), b, :, hs][0] for c in range(N_DEV)]
            vc = [vg_ref[pl.ds(slot[c], 1), b, :, hs][0] for c in range(N_DEV)]
            s0 = jnp.where(mask_a0, dot_t(qh, kc[0]), neg)
            s1 = jnp.where(mask_a1, dot_t(qh, kc[1][:128]), neg)
            sb0 = dot_t(q32, kc[1][128:])
            sb1 = dot_t(q32, kc[2])
            sb2 = dot_t(q32, kc[3])
            ma = jnp.maximum(jnp.max(s0, axis=1, keepdims=True),
                             jnp.max(s1, axis=1, keepdims=True))
            mb = jnp.maximum(jnp.max(sb0, axis=1, keepdims=True),
                             jnp.maximum(jnp.max(sb1, axis=1, keepdims=True),
                                         jnp.max(sb2, axis=1, keepdims=True)))
            m = jnp.maximum(ma, jnp.concatenate([mb, minf], axis=0))
            m32 = m[:GR]
            e0 = jnp.exp(s0 - m).astype(jnp.bfloat16)
            e1 = jnp.exp(s1 - m).astype(jnp.bfloat16)
            eb0 = jnp.exp(sb0 - m32).astype(jnp.bfloat16)
            eb1 = jnp.exp(sb1 - m32).astype(jnp.bfloat16)
            eb2 = jnp.exp(sb2 - m32).astype(jnp.bfloat16)
            db = (jnp.sum(eb0.astype(jnp.float32), axis=1, keepdims=True)
                  + jnp.sum(eb1.astype(jnp.float32), axis=1, keepdims=True)
                  + jnp.sum(eb2.astype(jnp.float32), axis=1, keepdims=True))
            denom = (jnp.sum(e0.astype(jnp.float32), axis=1, keepdims=True)
                     + jnp.sum(e1.astype(jnp.float32), axis=1, keepdims=True)
                     + jnp.concatenate([db, zcol], axis=0))
            na = dot_n(e0, vc[0]) + dot_n(e1, vc[1][:128])
            nb = (dot_n(eb0, vc[1][128:]) + dot_n(eb1, vc[2])
                  + dot_n(eb2, vc[3]))
            num = na + jnp.concatenate([nb, zctx], axis=0)
            ctx_ref[pl.ds(b * SQ, SQ), hs] = (num / denom).astype(jnp.bfloat16)

        bsl = pl.ds(b * SQ, SQ)
        partial_b = lax.dot_general(
            ctx_ref[bsl], wo_ref[...], (((1,), (0,)), ((), ())),
            preferred_element_type=jnp.float32,
        )
        myp_ref[bsl] = partial_b.astype(jnp.bfloat16)
        if not ABLATE_AR:
            for s in range(1, N_DEV):
                peer = (me + s) % N_DEV
                cp = pltpu.make_async_remote_copy(
                    src_ref=myp_ref.at[pl.ds(b * SQ + peer * QTR, QTR)],
                    dst_ref=rs_ref.at[b, s - 1],
                    send_sem=send_sems.at[6 + 3 * b + s - 1],
                    recv_sem=recv_sems.at[6 + 3 * b + s - 1],
                    device_id=(peer,),
                    device_id_type=pl.DeviceIdType.MESH,
                )
                cp.start()
                rs_copies.append(cp)

    if ABLATE_AR:
        out_ref[...] = myp_ref[...].astype(jnp.float32).reshape(B, SQ, DM)
        for cp in copies:
            cp.wait_send()
        return

    ag_copies = []
    for b in range(B):
        for t in range(3):
            rs_copies[3 * b + t].wait_recv()
        myq = myp_ref[pl.ds(b * SQ + me * QTR, QTR)].astype(jnp.float32)
        for t in range(3):
            myq = myq + rs_ref[b, t].astype(jnp.float32)
        ag_ref[b, 3] = myq.astype(jnp.bfloat16)
        out_ref[b, pl.ds(me * QTR, QTR)] = myq
        for s in range(1, N_DEV):
            peer = (me + s) % N_DEV
            cp = pltpu.make_async_remote_copy(
                src_ref=ag_ref.at[b, 3],
                dst_ref=ag_ref.at[b, s - 1],
                send_sem=send_sems.at[12 + 3 * b + s - 1],
                recv_sem=recv_sems.at[12 + 3 * b + s - 1],
                device_id=(peer,),
                device_id_type=pl.DeviceIdType.MESH,
            )
            cp.start()
            ag_copies.append(cp)

    for b in range(B):
        for s in range(1, N_DEV):
            ag_copies[3 * b + s - 1].wait_recv()
            origin = (me - s) % N_DEV
            out_ref[b, pl.ds(origin * QTR, QTR)] = (
                ag_ref[b, s - 1].astype(jnp.float32))

    for cp in copies:
        cp.wait_send()
    for cp in rs_copies + ag_copies:
        cp.wait_send()


def kernel(x, Wq, K_ext, V_ext, Wo):
    xb = x.astype(jnp.bfloat16)
    wqb = Wq.astype(jnp.bfloat16)
    wob = Wo.astype(jnp.bfloat16)
    kb = K_ext.reshape(B, SKV_LOC, HQ_G * DH).astype(jnp.bfloat16)
    vb = V_ext.reshape(B, SKV_LOC, HQ_G * DH).astype(jnp.bfloat16)

    out_shape = jax.ShapeDtypeStruct((B, SQ, DM), jnp.float32)
    return pl.pallas_call(
        _body,
        out_shape=out_shape,
        in_specs=[pl.BlockSpec(memory_space=pltpu.VMEM)] * 5,
        out_specs=pl.BlockSpec(memory_space=pltpu.VMEM),
        scratch_shapes=[
            pltpu.VMEM((N_DEV, B, SKV_LOC, DQ_LOC), jnp.bfloat16),
            pltpu.VMEM((N_DEV, B, SKV_LOC, DQ_LOC), jnp.bfloat16),
            pltpu.VMEM((B * SQ, DM), jnp.bfloat16),
            pltpu.VMEM((B, 3, QTR, DM), jnp.bfloat16),
            pltpu.VMEM((B, 4, QTR, DM), jnp.bfloat16),
            pltpu.VMEM((B * SQ, DQ_LOC), jnp.bfloat16),
            pltpu.VMEM((B * SQ, DQ_LOC), jnp.bfloat16),
            pltpu.SemaphoreType.DMA((18,)),
            pltpu.SemaphoreType.DMA((18,)),
        ],
        compiler_params=pltpu.CompilerParams(collective_id=0),
    )(xb, wqb, kb, vb, wob)
